# baseline (speedup 1.0000x reference)
import math

import jax
import jax.numpy as jnp
from jax.experimental import pallas as pl

N, C, H, W = 4, 256, 100, 352
HW = H * W
K = int(math.ceil(0.1 * HW))
WB = 128
NWB = (W + WB - 1) // WB
CELLS = H * WB


def _scores_kernel(x_ref, w1_ref, b1_ref, w2_ref, b2_ref, dis_ref, s_ref):
    xb = x_ref[0].reshape(C, CELLS)
    h = jax.lax.dot_general(w1_ref[...], xb, (((0,), (0,)), ((), ())))
    h = jnp.maximum(h + b1_ref[...].reshape(C, 1), 0.0)
    logits = jax.lax.dot_general(w2_ref[...], h, (((0,), (0,)), ((), ())))
    score = jax.nn.sigmoid(logits + b2_ref[...].reshape(1, 1))
    s_ref[0] = score.reshape(1, H, WB) * dis_ref[0]


def _thresh_kernel(s_ref, t_ref):
    bits = jax.lax.bitcast_convert_type(s_ref[...], jnp.int32)
    lo0 = jnp.zeros((N, 1, 1, 1), jnp.int32)
    hi0 = jnp.full((N, 1, 1, 1), 0x7F800000, jnp.int32)

    def body(_, carry):
        lo, hi = carry
        mid = lo + (hi - lo) // 2
        cnt = jnp.sum((bits >= mid).astype(jnp.int32), axis=(1, 2, 3),
                      keepdims=True)
        ge = cnt >= K
        return jnp.where(ge, mid, lo), jnp.where(ge, hi, mid)

    lo, _ = jax.lax.fori_loop(0, 31, body, (lo0, hi0))
    t_ref[...] = jax.lax.bitcast_convert_type(lo.reshape(N, 1), jnp.float32)


def _mask_kernel(x_ref, s_ref, t_ref, o_ref):
    n = pl.program_id(0)
    t = t_ref[pl.ds(n, 1), :]
    s = s_ref[0]
    keep = (s >= t.reshape(1, 1, 1)).astype(jnp.float32)
    o_ref[0] = x_ref[0] * (s * keep)


def kernel(spatial_features_2d, dis_priority, w1, b1, w2, b2):
    x = spatial_features_2d
    dis = dis_priority.reshape(N, 1, H, W)
    b1r = b1.reshape(1, C)
    b2r = b2.reshape(1, 1)

    scores = pl.pallas_call(
        _scores_kernel,
        grid=(N, NWB),
        in_specs=[
            pl.BlockSpec((1, C, H, WB), lambda n, w: (n, 0, 0, w)),
            pl.BlockSpec((C, C), lambda n, w: (0, 0)),
            pl.BlockSpec((1, C), lambda n, w: (0, 0)),
            pl.BlockSpec((C, 1), lambda n, w: (0, 0)),
            pl.BlockSpec((1, 1), lambda n, w: (0, 0)),
            pl.BlockSpec((1, 1, H, WB), lambda n, w: (n, 0, 0, w)),
        ],
        out_specs=pl.BlockSpec((1, 1, H, WB), lambda n, w: (n, 0, 0, w)),
        out_shape=jax.ShapeDtypeStruct((N, 1, H, W), jnp.float32),
    )(x, w1, b1r, w2, b2r, dis)

    thresh = pl.pallas_call(
        _thresh_kernel,
        out_shape=jax.ShapeDtypeStruct((N, 1), jnp.float32),
    )(scores)

    CB = C // 2
    out = pl.pallas_call(
        _mask_kernel,
        grid=(N, 2, NWB),
        in_specs=[
            pl.BlockSpec((1, CB, H, WB), lambda n, c, w: (n, c, 0, w)),
            pl.BlockSpec((1, 1, H, WB), lambda n, c, w: (n, 0, 0, w)),
            pl.BlockSpec((N, 1), lambda n, c, w: (0, 0)),
        ],
        out_specs=pl.BlockSpec((1, CB, H, WB), lambda n, c, w: (n, c, 0, w)),
        out_shape=jax.ShapeDtypeStruct((N, C, H, W), jnp.float32),
    )(x, scores, thresh)

    return out


# speedup vs baseline: 2.5950x; 2.5950x over previous
import math

import jax
import jax.numpy as jnp
from jax.experimental import pallas as pl

N, C, H, W = 4, 256, 100, 352
HW = H * W
K = int(math.ceil(0.1 * HW))
BLK = 7040
NB = HW // BLK


def _scores_kernel(x_ref, w1_ref, b1_ref, w2_ref, b2_ref, dis_ref, s_ref):
    xm = x_ref[0]
    h = jax.lax.dot_general(xm, w1_ref[...], (((1,), (0,)), ((), ())))
    h = jnp.maximum(h + b1_ref[...], 0.0)
    logits = jax.lax.dot_general(w2_ref[...], h, (((0,), (1,)), ((), ())))
    score = jax.nn.sigmoid(logits + b2_ref[...])
    s_ref[0] = score * dis_ref[0]


def _thresh_kernel(s_ref, t_ref):
    bits = jax.lax.bitcast_convert_type(s_ref[...], jnp.int32)
    lo0 = jnp.zeros((N, 1, 1), jnp.int32)
    hi0 = jnp.full((N, 1, 1), 0x7F800000, jnp.int32)

    def body(_, carry):
        lo, hi = carry
        mid = lo + (hi - lo) // 2
        cnt = jnp.sum((bits >= mid).astype(jnp.int32), axis=(1, 2),
                      keepdims=True)
        ge = cnt >= K
        return jnp.where(ge, mid, lo), jnp.where(ge, hi, mid)

    lo, _ = jax.lax.fori_loop(0, 31, body, (lo0, hi0))
    t_ref[...] = jax.lax.bitcast_convert_type(lo.reshape(N, 1), jnp.float32)


def _mask_kernel(x_ref, s_ref, t_ref, o_ref):
    n = pl.program_id(0)
    t = t_ref[pl.ds(n, 1), :]
    s = s_ref[0]
    sk = s * (s >= t).astype(jnp.float32)
    o_ref[0] = x_ref[0] * sk.reshape(BLK, 1)


def kernel(spatial_features_2d, dis_priority, w1, b1, w2, b2):
    xt = spatial_features_2d.transpose(0, 2, 3, 1).reshape(N, HW, C)
    dis = dis_priority.reshape(N, 1, HW)
    b1r = b1.reshape(1, C)
    b2r = b2.reshape(1, 1)

    scores = pl.pallas_call(
        _scores_kernel,
        grid=(N, NB),
        in_specs=[
            pl.BlockSpec((1, BLK, C), lambda n, b: (n, b, 0)),
            pl.BlockSpec((C, C), lambda n, b: (0, 0)),
            pl.BlockSpec((1, C), lambda n, b: (0, 0)),
            pl.BlockSpec((C, 1), lambda n, b: (0, 0)),
            pl.BlockSpec((1, 1), lambda n, b: (0, 0)),
            pl.BlockSpec((1, 1, BLK), lambda n, b: (n, 0, b)),
        ],
        out_specs=pl.BlockSpec((1, 1, BLK), lambda n, b: (n, 0, b)),
        out_shape=jax.ShapeDtypeStruct((N, 1, HW), jnp.float32),
    )(xt, w1, b1r, w2, b2r, dis)

    thresh = pl.pallas_call(
        _thresh_kernel,
        out_shape=jax.ShapeDtypeStruct((N, 1), jnp.float32),
    )(scores)

    out_t = pl.pallas_call(
        _mask_kernel,
        grid=(N, NB),
        in_specs=[
            pl.BlockSpec((1, BLK, C), lambda n, b: (n, b, 0)),
            pl.BlockSpec((1, 1, BLK), lambda n, b: (n, 0, b)),
            pl.BlockSpec((N, 1), lambda n, b: (0, 0)),
        ],
        out_specs=pl.BlockSpec((1, BLK, C), lambda n, b: (n, b, 0)),
        out_shape=jax.ShapeDtypeStruct((N, HW, C), jnp.float32),
    )(xt, scores, thresh)

    return out_t.reshape(N, H, W, C).transpose(0, 3, 1, 2)
